# baseline (speedup 1.0000x reference)
import functools

import jax
import jax.numpy as jnp
from jax import lax
from jax.experimental import pallas as pl
from jax.experimental.pallas import tpu as pltpu
from jax.experimental.pallas import tpu_sc as plsc

_D = 16
_D_IN = 32
_K = 512
_D_OUT = 8
_BN_EPS = 1e-3
_LOG2E = 1.4426950408889634
_HALF = _D // 2
_WIDE = _HALF * _K
_KGRP = _HALF * _D_IN


def _sc_gather(table, idx):
    B = idx.shape[0]
    Dw = table.shape[1]
    NW = 32
    b_per_w = B // NW
    C = 128
    n_chunks = b_per_w // C
    mesh = plsc.VectorSubcoreMesh(core_axis_name="c", subcore_axis_name="s")

    @functools.partial(
        pl.kernel,
        mesh=mesh,
        out_type=jax.ShapeDtypeStruct((B, Dw), jnp.float32),
        scratch_types=[
            pltpu.VMEM((C,), jnp.int32),
            pltpu.VMEM((C, Dw), jnp.float32),
            pltpu.SemaphoreType.DMA,
        ],
    )
    def k(table_hbm, idx_hbm, out_hbm, idx_v, rows_v, sem):
        wid = lax.axis_index("s") * 2 + lax.axis_index("c")
        base = wid * b_per_w
        for c in range(n_chunks):
            off = base + c * C
            pltpu.sync_copy(idx_hbm.at[pl.ds(off, C)], idx_v)
            pltpu.async_copy(table_hbm.at[idx_v], rows_v, sem).wait()
            pltpu.sync_copy(rows_v, out_hbm.at[pl.ds(off, C)])

    return k(table, idx)


def _stats_body(x_ref, cktile_ref, bd_ref, wsraw_ref, cva_ref, g_ref, bt_ref,
                ws_ref, cvs_ref, cs_ref, p_ref, *, inv_n, nb):
    j = pl.program_id(0)
    xb = x_ref[...]
    cs = jnp.sum(xb, axis=0, keepdims=True)
    P = lax.dot_general(xb, xb, (((0,), (0,)), ((), ())),
                        preferred_element_type=jnp.float32)

    @pl.when(j == 0)
    def _():
        cs_ref[...] = cs
        p_ref[...] = P

    @pl.when(j > 0)
    def _():
        cs_ref[...] = cs_ref[...] + cs
        p_ref[...] = p_ref[...] + P

    @pl.when(j == nb - 1)
    def _():
        cktile = cktile_ref[...]
        mean = lax.dot_general(cktile, cs_ref[...], (((0,), (1,)), ((), ())),
                               preferred_element_type=jnp.float32) * inv_n
        pd = p_ref[...] * bd_ref[...]
        H = lax.dot_general(pd, cktile, (((1,), (0,)), ((), ())),
                            preferred_element_type=jnp.float32)
        ones_row = jnp.ones((1, _K), jnp.float32)
        ex2 = lax.dot_general(cktile * H, ones_row, (((0,), (1,)), ((), ())),
                              preferred_element_type=jnp.float32) * inv_n
        var = ex2 - mean * mean
        a_col = g_ref[...] * lax.rsqrt(var + _BN_EPS)
        b2_col = (bt_ref[...] - mean * a_col) * _LOG2E
        a2_col = a_col * _LOG2E
        cvs_ref[...] = cva_ref[...] * jnp.exp2(b2_col)
        a_t = jnp.concatenate([a2_col] * _HALF, axis=0)
        ws_ref[...] = (wsraw_ref[...] * a_t).astype(jnp.bfloat16)


def _apply_body(x_ref, ws_ref, cvs_ref, shuf_ref, out_ref):
    xb = x_ref[...].astype(jnp.bfloat16)
    ws = ws_ref[...]
    cvs = cvs_ref[...]
    nt = (((1,), (1,)), ((), ()))
    tn = (((0,), (0,)), ((), ()))
    pieces = []
    for half in range(2):
        xh = xb[:, half * _KGRP:(half + 1) * _KGRP]
        for c in range(_HALF):
            RcT = lax.dot_general(ws[c * _K:(c + 1) * _K, :], xh, nt,
                                  preferred_element_type=jnp.float32)
            EcT = jnp.exp2(RcT)
            pieces.append(lax.dot_general(cvs, EcT, tn,
                                          preferred_element_type=jnp.float32))
    yt = jnp.concatenate(pieces, axis=0)
    nd = lax.dot_general(yt, shuf_ref[...], tn,
                         preferred_element_type=jnp.float32)
    out_ref[...] = nd[:, 0:128] / nd[:, 128:256]


def _tc_main(x2, cktile, bd, wsraw, cva, shuf, gamma, beta):
    N20 = x2.shape[0]
    N = N20 * _D
    MS = 2048
    NBS = N20 // MS
    M20 = 256
    NB = N20 // M20
    stats = functools.partial(_stats_body, inv_n=float(1.0 / N), nb=NBS)
    ws, cvs = pl.pallas_call(
        stats,
        grid=(NBS,),
        in_specs=[
            pl.BlockSpec((MS, _K), lambda j: (j, 0)),
            pl.BlockSpec((_K, _K), lambda j: (0, 0)),
            pl.BlockSpec((_K, _K), lambda j: (0, 0)),
            pl.BlockSpec((_WIDE, _KGRP), lambda j: (0, 0)),
            pl.BlockSpec((_K, _D), lambda j: (0, 0)),
            pl.BlockSpec((_K, 1), lambda j: (0, 0)),
            pl.BlockSpec((_K, 1), lambda j: (0, 0)),
        ],
        out_specs=[
            pl.BlockSpec((_WIDE, _KGRP), lambda j: (0, 0)),
            pl.BlockSpec((_K, _D), lambda j: (0, 0)),
        ],
        out_shape=[
            jax.ShapeDtypeStruct((_WIDE, _KGRP), jnp.bfloat16),
            jax.ShapeDtypeStruct((_K, _D), jnp.float32),
        ],
        scratch_shapes=[
            pltpu.VMEM((1, _K), jnp.float32),
            pltpu.VMEM((_K, _K), jnp.float32),
        ],
        compiler_params=pltpu.CompilerParams(
            dimension_semantics=("arbitrary",),
        ),
    )(x2, cktile, bd, wsraw, cva, gamma, beta)
    return pl.pallas_call(
        _apply_body,
        grid=(NB,),
        in_specs=[
            pl.BlockSpec((M20, _K), lambda j: (j, 0)),
            pl.BlockSpec((_WIDE, _KGRP), lambda j: (0, 0)),
            pl.BlockSpec((_K, _D), lambda j: (0, 0)),
            pl.BlockSpec((2 * _D * _D_OUT, 2 * _D * _D_OUT), lambda j: (0, 0)),
        ],
        out_specs=pl.BlockSpec((M20, _D * _D_OUT), lambda j: (j, 0)),
        out_shape=jax.ShapeDtypeStruct((N20, _D * _D_OUT), jnp.float32),
        compiler_params=pltpu.CompilerParams(
            dimension_semantics=("arbitrary",),
        ),
    )(x2, ws, cvs, shuf)


def _build_constants(centroids_k, centroids_v):
    ckT = centroids_k.T
    cktile = jnp.tile(ckT, (_D, 1))
    eye16 = jnp.eye(_D, dtype=jnp.float32)
    bd = jnp.kron(eye16, jnp.ones((_D_IN, _D_IN), jnp.float32))
    eye8 = jnp.eye(_HALF, dtype=jnp.float32)
    wsraw = jnp.reshape(
        jnp.einsum('ce,ki->ckei', eye8, centroids_k), (_WIDE, _KGRP))
    cva = jnp.concatenate(
        [centroids_v, jnp.ones((_K, 1), jnp.float32),
         jnp.zeros((_K, _D - _D_OUT - 1), jnp.float32)], axis=1)
    numpart = jnp.reshape(
        jnp.einsum('de,uv->duev', eye16, jnp.eye(_D, _D_OUT)),
        (_D * _D, _D * _D_OUT))
    denpart = jnp.reshape(
        jnp.einsum('de,u,v->duev', eye16,
                   (jnp.arange(_D) == _D_OUT).astype(jnp.float32),
                   jnp.ones((_D_OUT,), jnp.float32)),
        (_D * _D, _D * _D_OUT))
    shuf = jnp.concatenate([numpart, denpart], axis=1)
    return cktile, bd, wsraw, cva, shuf


def kernel(input, query_wemb, centroids_k, centroids_v, bn_gamma, bn_beta):
    idxs = jnp.reshape(input, (-1,))
    x2 = _sc_gather(query_wemb, idxs)
    cktile, bd, wsraw, cva, shuf = _build_constants(centroids_k, centroids_v)
    out128 = _tc_main(x2, cktile, bd, wsraw, cva, shuf,
                      jnp.reshape(bn_gamma, (_K, 1)),
                      jnp.reshape(bn_beta, (_K, 1)))
    out = jnp.reshape(out128, tuple(input.shape) + (_D * _D_OUT,))
    losses = jnp.zeros((), dtype=jnp.float32)
    return (out, losses)


# speedup vs baseline: 1.7678x; 1.7678x over previous
import functools

import jax
import jax.numpy as jnp
from jax import lax
from jax.experimental import pallas as pl
from jax.experimental.pallas import tpu as pltpu
from jax.experimental.pallas import tpu_sc as plsc

_D = 16
_D_IN = 32
_K = 512
_D_OUT = 8
_BN_EPS = 1e-3
_LOG2E = 1.4426950408889634
_HALF = _D // 2
_WIDE = _HALF * _K
_KGRP = _HALF * _D_IN


def _sc_gather(table, idx):
    B = idx.shape[0]
    Dw = table.shape[1]
    NW = 32
    b_per_w = B // NW
    C = 128
    n_chunks = b_per_w // C
    mesh = plsc.VectorSubcoreMesh(core_axis_name="c", subcore_axis_name="s")

    @functools.partial(
        pl.kernel,
        mesh=mesh,
        out_type=jax.ShapeDtypeStruct((B, Dw), jnp.float32),
        scratch_types=[
            pltpu.VMEM((C,), jnp.int32),
            pltpu.VMEM((C, Dw), jnp.float32),
            pltpu.SemaphoreType.DMA,
        ],
    )
    def k(table_hbm, idx_hbm, out_hbm, idx_v, rows_v, sem):
        wid = lax.axis_index("s") * 2 + lax.axis_index("c")
        base = wid * b_per_w
        for c in range(n_chunks):
            off = base + c * C
            pltpu.sync_copy(idx_hbm.at[pl.ds(off, C)], idx_v)
            pltpu.async_copy(table_hbm.at[idx_v], rows_v, sem).wait()
            pltpu.sync_copy(rows_v, out_hbm.at[pl.ds(off, C)])

    return k(table, idx)


def _stats_body(x_ref, cktile_ref, bd_ref, wsraw_ref, cva_ref, g_ref, bt_ref,
                ws_ref, cvs_ref, cs_ref, p_ref, *, inv_n, nb):
    j = pl.program_id(0)
    xb = x_ref[...]
    cs = jnp.sum(xb, axis=0, keepdims=True)
    P = lax.dot_general(xb, xb, (((0,), (0,)), ((), ())),
                        preferred_element_type=jnp.float32)

    @pl.when(j == 0)
    def _():
        cs_ref[...] = cs
        p_ref[...] = P

    @pl.when(j > 0)
    def _():
        cs_ref[...] = cs_ref[...] + cs
        p_ref[...] = p_ref[...] + P

    @pl.when(j == nb - 1)
    def _():
        cktile = cktile_ref[...]
        mean = lax.dot_general(cktile, cs_ref[...], (((0,), (1,)), ((), ())),
                               preferred_element_type=jnp.float32) * inv_n
        pd = p_ref[...] * bd_ref[...]
        H = lax.dot_general(pd, cktile, (((1,), (0,)), ((), ())),
                            preferred_element_type=jnp.float32)
        ones_row = jnp.ones((1, _K), jnp.float32)
        ex2 = lax.dot_general(cktile * H, ones_row, (((0,), (1,)), ((), ())),
                              preferred_element_type=jnp.float32) * inv_n
        var = ex2 - mean * mean
        a_col = g_ref[...] * lax.rsqrt(var + _BN_EPS)
        b2_col = (bt_ref[...] - mean * a_col) * _LOG2E
        a2_col = a_col * _LOG2E
        cvs_ref[...] = cva_ref[...] * jnp.exp2(b2_col)
        a_t = jnp.concatenate([a2_col] * _HALF, axis=0)
        ws_ref[...] = (wsraw_ref[...] * a_t).astype(jnp.bfloat16)


def _apply_body(x_ref, ws_ref, cvs_ref, shuf_ref, out_ref):
    xb = x_ref[...].astype(jnp.bfloat16)
    ws = ws_ref[...]
    cvs = cvs_ref[...]
    nt = (((1,), (1,)), ((), ()))
    tn = (((0,), (0,)), ((), ()))
    EaT = jnp.exp2(lax.dot_general(ws, xb[:, 0:_KGRP], nt,
                                   preferred_element_type=jnp.float32))
    EbT = jnp.exp2(lax.dot_general(ws, xb[:, _KGRP:2 * _KGRP], nt,
                                   preferred_element_type=jnp.float32))
    pieces = []
    for c in range(_HALF):
        pieces.append(lax.dot_general(cvs, EaT[c * _K:(c + 1) * _K, :], tn,
                                      preferred_element_type=jnp.float32))
    for c in range(_HALF):
        pieces.append(lax.dot_general(cvs, EbT[c * _K:(c + 1) * _K, :], tn,
                                      preferred_element_type=jnp.float32))
    yt = jnp.concatenate(pieces, axis=0)
    nd = lax.dot_general(yt, shuf_ref[...], tn,
                         preferred_element_type=jnp.float32)
    out_ref[...] = nd[:, 0:128] / nd[:, 128:256]


def _tc_main(x2, cktile, bd, wsraw, cva, shuf, gamma, beta):
    N20 = x2.shape[0]
    N = N20 * _D
    MS = 2048
    NBS = N20 // MS
    M20 = 512
    NB = N20 // M20
    stats = functools.partial(_stats_body, inv_n=float(1.0 / N), nb=NBS)
    ws, cvs = pl.pallas_call(
        stats,
        grid=(NBS,),
        in_specs=[
            pl.BlockSpec((MS, _K), lambda j: (j, 0)),
            pl.BlockSpec((_K, _K), lambda j: (0, 0)),
            pl.BlockSpec((_K, _K), lambda j: (0, 0)),
            pl.BlockSpec((_WIDE, _KGRP), lambda j: (0, 0)),
            pl.BlockSpec((_K, _D), lambda j: (0, 0)),
            pl.BlockSpec((_K, 1), lambda j: (0, 0)),
            pl.BlockSpec((_K, 1), lambda j: (0, 0)),
        ],
        out_specs=[
            pl.BlockSpec((_WIDE, _KGRP), lambda j: (0, 0)),
            pl.BlockSpec((_K, _D), lambda j: (0, 0)),
        ],
        out_shape=[
            jax.ShapeDtypeStruct((_WIDE, _KGRP), jnp.bfloat16),
            jax.ShapeDtypeStruct((_K, _D), jnp.float32),
        ],
        scratch_shapes=[
            pltpu.VMEM((1, _K), jnp.float32),
            pltpu.VMEM((_K, _K), jnp.float32),
        ],
        compiler_params=pltpu.CompilerParams(
            dimension_semantics=("arbitrary",),
        ),
    )(x2, cktile, bd, wsraw, cva, gamma, beta)
    return pl.pallas_call(
        _apply_body,
        grid=(NB,),
        in_specs=[
            pl.BlockSpec((M20, _K), lambda j: (j, 0)),
            pl.BlockSpec((_WIDE, _KGRP), lambda j: (0, 0)),
            pl.BlockSpec((_K, _D), lambda j: (0, 0)),
            pl.BlockSpec((2 * _D * _D_OUT, 2 * _D * _D_OUT), lambda j: (0, 0)),
        ],
        out_specs=pl.BlockSpec((M20, _D * _D_OUT), lambda j: (j, 0)),
        out_shape=jax.ShapeDtypeStruct((N20, _D * _D_OUT), jnp.float32),
        compiler_params=pltpu.CompilerParams(
            dimension_semantics=("arbitrary",),
        ),
    )(x2, ws, cvs, shuf)


def _build_constants(centroids_k, centroids_v):
    ckT = centroids_k.T
    cktile = jnp.tile(ckT, (_D, 1))
    eye16 = jnp.eye(_D, dtype=jnp.float32)
    bd = jnp.kron(eye16, jnp.ones((_D_IN, _D_IN), jnp.float32))
    eye8 = jnp.eye(_HALF, dtype=jnp.float32)
    wsraw = jnp.reshape(
        jnp.einsum('ce,ki->ckei', eye8, centroids_k), (_WIDE, _KGRP))
    cva = jnp.concatenate(
        [centroids_v, jnp.ones((_K, 1), jnp.float32),
         jnp.zeros((_K, _D - _D_OUT - 1), jnp.float32)], axis=1)
    numpart = jnp.reshape(
        jnp.einsum('de,uv->duev', eye16, jnp.eye(_D, _D_OUT)),
        (_D * _D, _D * _D_OUT))
    denpart = jnp.reshape(
        jnp.einsum('de,u,v->duev', eye16,
                   (jnp.arange(_D) == _D_OUT).astype(jnp.float32),
                   jnp.ones((_D_OUT,), jnp.float32)),
        (_D * _D, _D * _D_OUT))
    shuf = jnp.concatenate([numpart, denpart], axis=1)
    return cktile, bd, wsraw, cva, shuf


def kernel(input, query_wemb, centroids_k, centroids_v, bn_gamma, bn_beta):
    idxs = jnp.reshape(input, (-1,))
    x2 = _sc_gather(query_wemb, idxs)
    cktile, bd, wsraw, cva, shuf = _build_constants(centroids_k, centroids_v)
    out128 = _tc_main(x2, cktile, bd, wsraw, cva, shuf,
                      jnp.reshape(bn_gamma, (_K, 1)),
                      jnp.reshape(bn_beta, (_K, 1)))
    out = jnp.reshape(out128, tuple(input.shape) + (_D * _D_OUT,))
    losses = jnp.zeros((), dtype=jnp.float32)
    return (out, losses)


# speedup vs baseline: 1.8068x; 1.0220x over previous
import functools

import jax
import jax.numpy as jnp
from jax import lax
from jax.experimental import pallas as pl
from jax.experimental.pallas import tpu as pltpu
from jax.experimental.pallas import tpu_sc as plsc

_D = 16
_D_IN = 32
_K = 512
_D_OUT = 8
_BN_EPS = 1e-3
_LOG2E = 1.4426950408889634
_HALF = _D // 2
_WIDE = _HALF * _K
_KGRP = _HALF * _D_IN


def _sc_gather(table, idx):
    B = idx.shape[0]
    Dw = table.shape[1]
    NW = 32
    b_per_w = B // NW
    C = 128
    n_chunks = b_per_w // C
    mesh = plsc.VectorSubcoreMesh(core_axis_name="c", subcore_axis_name="s")

    @functools.partial(
        pl.kernel,
        mesh=mesh,
        out_type=jax.ShapeDtypeStruct((B, Dw), jnp.float32),
        scratch_types=[
            pltpu.VMEM((C,), jnp.int32),
            pltpu.VMEM((C, Dw), jnp.float32),
            pltpu.SemaphoreType.DMA,
        ],
    )
    def k(table_hbm, idx_hbm, out_hbm, idx_v, rows_v, sem):
        wid = lax.axis_index("s") * 2 + lax.axis_index("c")
        base = wid * b_per_w
        for c in range(n_chunks):
            off = base + c * C
            pltpu.sync_copy(idx_hbm.at[pl.ds(off, C)], idx_v)
            pltpu.async_copy(table_hbm.at[idx_v], rows_v, sem).wait()
            pltpu.sync_copy(rows_v, out_hbm.at[pl.ds(off, C)])

    return k(table, idx)


def _stats_body(x_ref, cktile_ref, bd_ref, wsraw_ref, cva_ref, g_ref, bt_ref,
                ws_ref, cvs_ref, cs_ref, p_ref, *, inv_n, nb):
    j = pl.program_id(0)
    xb = x_ref[...]
    cs = jnp.sum(xb, axis=0, keepdims=True)
    P = lax.dot_general(xb, xb, (((0,), (0,)), ((), ())),
                        preferred_element_type=jnp.float32)

    @pl.when(j == 0)
    def _():
        cs_ref[...] = cs
        p_ref[...] = P

    @pl.when(j > 0)
    def _():
        cs_ref[...] = cs_ref[...] + cs
        p_ref[...] = p_ref[...] + P

    @pl.when(j == nb - 1)
    def _():
        cktile = cktile_ref[...]
        mean = lax.dot_general(cktile, cs_ref[...], (((0,), (1,)), ((), ())),
                               preferred_element_type=jnp.float32) * inv_n
        pd = p_ref[...] * bd_ref[...]
        H = lax.dot_general(pd, cktile, (((1,), (0,)), ((), ())),
                            preferred_element_type=jnp.float32)
        ones_row = jnp.ones((1, _K), jnp.float32)
        ex2 = lax.dot_general(cktile * H, ones_row, (((0,), (1,)), ((), ())),
                              preferred_element_type=jnp.float32) * inv_n
        var = ex2 - mean * mean
        a_col = g_ref[...] * lax.rsqrt(var + _BN_EPS)
        b2_col = (bt_ref[...] - mean * a_col) * _LOG2E
        a2_col = a_col * _LOG2E
        cvs_ref[...] = cva_ref[...] * jnp.exp2(b2_col)
        a_t = jnp.concatenate([a2_col] * _HALF, axis=0)
        ws_ref[...] = (wsraw_ref[...] * a_t).astype(jnp.bfloat16)


def _apply_body(x_ref, ws_ref, cvs_ref, shuf_ref, out_ref):
    xb = x_ref[...].astype(jnp.bfloat16)
    ws = ws_ref[...]
    cvs = cvs_ref[...]
    nt = (((1,), (1,)), ((), ()))
    tn = (((0,), (0,)), ((), ()))
    EaT = jnp.exp2(lax.dot_general(ws, xb[:, 0:_KGRP], nt,
                                   preferred_element_type=jnp.float32))
    EbT = jnp.exp2(lax.dot_general(ws, xb[:, _KGRP:2 * _KGRP], nt,
                                   preferred_element_type=jnp.float32))
    pieces = []
    for c in range(_HALF):
        pieces.append(lax.dot_general(cvs, EaT[c * _K:(c + 1) * _K, :], tn,
                                      preferred_element_type=jnp.float32))
    for c in range(_HALF):
        pieces.append(lax.dot_general(cvs, EbT[c * _K:(c + 1) * _K, :], tn,
                                      preferred_element_type=jnp.float32))
    yt = jnp.concatenate(pieces, axis=0)
    nd = lax.dot_general(yt, shuf_ref[...], tn,
                         preferred_element_type=jnp.float32)
    out_ref[...] = nd[:, 0:128] / nd[:, 128:256]


def _tc_main(x2, cktile, bd, wsraw, cva, shuf, gamma, beta):
    N20 = x2.shape[0]
    N = N20 * _D
    MS = 2048
    NBS = N20 // MS
    M20 = 1024
    NB = N20 // M20
    stats = functools.partial(_stats_body, inv_n=float(1.0 / N), nb=NBS)
    ws, cvs = pl.pallas_call(
        stats,
        grid=(NBS,),
        in_specs=[
            pl.BlockSpec((MS, _K), lambda j: (j, 0)),
            pl.BlockSpec((_K, _K), lambda j: (0, 0)),
            pl.BlockSpec((_K, _K), lambda j: (0, 0)),
            pl.BlockSpec((_WIDE, _KGRP), lambda j: (0, 0)),
            pl.BlockSpec((_K, _D), lambda j: (0, 0)),
            pl.BlockSpec((_K, 1), lambda j: (0, 0)),
            pl.BlockSpec((_K, 1), lambda j: (0, 0)),
        ],
        out_specs=[
            pl.BlockSpec((_WIDE, _KGRP), lambda j: (0, 0)),
            pl.BlockSpec((_K, _D), lambda j: (0, 0)),
        ],
        out_shape=[
            jax.ShapeDtypeStruct((_WIDE, _KGRP), jnp.bfloat16),
            jax.ShapeDtypeStruct((_K, _D), jnp.float32),
        ],
        scratch_shapes=[
            pltpu.VMEM((1, _K), jnp.float32),
            pltpu.VMEM((_K, _K), jnp.float32),
        ],
        compiler_params=pltpu.CompilerParams(
            dimension_semantics=("arbitrary",),
        ),
    )(x2, cktile, bd, wsraw, cva, gamma, beta)
    return pl.pallas_call(
        _apply_body,
        grid=(NB,),
        in_specs=[
            pl.BlockSpec((M20, _K), lambda j: (j, 0)),
            pl.BlockSpec((_WIDE, _KGRP), lambda j: (0, 0)),
            pl.BlockSpec((_K, _D), lambda j: (0, 0)),
            pl.BlockSpec((2 * _D * _D_OUT, 2 * _D * _D_OUT), lambda j: (0, 0)),
        ],
        out_specs=pl.BlockSpec((M20, _D * _D_OUT), lambda j: (j, 0)),
        out_shape=jax.ShapeDtypeStruct((N20, _D * _D_OUT), jnp.float32),
        compiler_params=pltpu.CompilerParams(
            dimension_semantics=("arbitrary",),
        ),
    )(x2, ws, cvs, shuf)


def _build_constants(centroids_k, centroids_v):
    ckT = centroids_k.T
    cktile = jnp.tile(ckT, (_D, 1))
    eye16 = jnp.eye(_D, dtype=jnp.float32)
    bd = jnp.kron(eye16, jnp.ones((_D_IN, _D_IN), jnp.float32))
    eye8 = jnp.eye(_HALF, dtype=jnp.float32)
    wsraw = jnp.reshape(
        jnp.einsum('ce,ki->ckei', eye8, centroids_k), (_WIDE, _KGRP))
    cva = jnp.concatenate(
        [centroids_v, jnp.ones((_K, 1), jnp.float32),
         jnp.zeros((_K, _D - _D_OUT - 1), jnp.float32)], axis=1)
    numpart = jnp.reshape(
        jnp.einsum('de,uv->duev', eye16, jnp.eye(_D, _D_OUT)),
        (_D * _D, _D * _D_OUT))
    denpart = jnp.reshape(
        jnp.einsum('de,u,v->duev', eye16,
                   (jnp.arange(_D) == _D_OUT).astype(jnp.float32),
                   jnp.ones((_D_OUT,), jnp.float32)),
        (_D * _D, _D * _D_OUT))
    shuf = jnp.concatenate([numpart, denpart], axis=1)
    return cktile, bd, wsraw, cva, shuf


def kernel(input, query_wemb, centroids_k, centroids_v, bn_gamma, bn_beta):
    idxs = jnp.reshape(input, (-1,))
    x2 = _sc_gather(query_wemb, idxs)
    cktile, bd, wsraw, cva, shuf = _build_constants(centroids_k, centroids_v)
    out128 = _tc_main(x2, cktile, bd, wsraw, cva, shuf,
                      jnp.reshape(bn_gamma, (_K, 1)),
                      jnp.reshape(bn_beta, (_K, 1)))
    out = jnp.reshape(out128, tuple(input.shape) + (_D * _D_OUT,))
    losses = jnp.zeros((), dtype=jnp.float32)
    return (out, losses)


# speedup vs baseline: 1.8130x; 1.0035x over previous
import functools

import jax
import jax.numpy as jnp
from jax import lax
from jax.experimental import pallas as pl
from jax.experimental.pallas import tpu as pltpu
from jax.experimental.pallas import tpu_sc as plsc

_D = 16
_D_IN = 32
_K = 512
_D_OUT = 8
_BN_EPS = 1e-3
_LOG2E = 1.4426950408889634
_HALF = _D // 2
_WIDE = _HALF * _K
_KGRP = _HALF * _D_IN


def _sc_gather(table, idx):
    B = idx.shape[0]
    Dw = table.shape[1]
    NW = 32
    b_per_w = B // NW
    C = 128
    n_chunks = b_per_w // C
    mesh = plsc.VectorSubcoreMesh(core_axis_name="c", subcore_axis_name="s")

    @functools.partial(
        pl.kernel,
        mesh=mesh,
        out_type=jax.ShapeDtypeStruct((B, Dw), jnp.float32),
        scratch_types=[
            pltpu.VMEM((C,), jnp.int32),
            pltpu.VMEM((C, Dw), jnp.float32),
            pltpu.SemaphoreType.DMA,
        ],
    )
    def k(table_hbm, idx_hbm, out_hbm, idx_v, rows_v, sem):
        wid = lax.axis_index("s") * 2 + lax.axis_index("c")
        base = wid * b_per_w
        for c in range(n_chunks):
            off = base + c * C
            pltpu.sync_copy(idx_hbm.at[pl.ds(off, C)], idx_v)
            pltpu.async_copy(table_hbm.at[idx_v], rows_v, sem).wait()
            pltpu.sync_copy(rows_v, out_hbm.at[pl.ds(off, C)])

    return k(table, idx)


def _stats_body(x_ref, cktile_ref, bd_ref, wsraw_ref, cva_ref, g_ref, bt_ref,
                ws_ref, cvs_ref, cs_ref, p_ref, *, inv_n, nb):
    j = pl.program_id(0)
    xb = x_ref[...]
    cs = jnp.sum(xb, axis=0, keepdims=True)
    P = lax.dot_general(xb, xb, (((0,), (0,)), ((), ())),
                        preferred_element_type=jnp.float32)

    @pl.when(j == 0)
    def _():
        cs_ref[...] = cs
        p_ref[...] = P

    @pl.when(j > 0)
    def _():
        cs_ref[...] = cs_ref[...] + cs
        p_ref[...] = p_ref[...] + P

    @pl.when(j == nb - 1)
    def _():
        cktile = cktile_ref[...]
        mean = lax.dot_general(cktile, cs_ref[...], (((0,), (1,)), ((), ())),
                               preferred_element_type=jnp.float32) * inv_n
        pd = p_ref[...] * bd_ref[...]
        H = lax.dot_general(pd, cktile, (((1,), (0,)), ((), ())),
                            preferred_element_type=jnp.float32)
        ones_row = jnp.ones((1, _K), jnp.float32)
        ex2 = lax.dot_general(cktile * H, ones_row, (((0,), (1,)), ((), ())),
                              preferred_element_type=jnp.float32) * inv_n
        var = ex2 - mean * mean
        a_col = g_ref[...] * lax.rsqrt(var + _BN_EPS)
        b2_col = (bt_ref[...] - mean * a_col) * _LOG2E
        a2_col = a_col * _LOG2E
        cvs_ref[...] = cva_ref[...] * jnp.exp2(b2_col)
        a_t = jnp.concatenate([a2_col] * _HALF, axis=0)
        ws_ref[...] = (wsraw_ref[...] * a_t).astype(jnp.bfloat16)


def _apply_body(x_ref, ws_ref, cvs_ref, shuf_ref, out_ref):
    xb = x_ref[...].astype(jnp.bfloat16)
    ws = ws_ref[...]
    cvs = cvs_ref[...]
    nt = (((1,), (1,)), ((), ()))
    tn = (((0,), (0,)), ((), ()))
    EaT = jnp.exp2(lax.dot_general(ws, xb[:, 0:_KGRP], nt,
                                   preferred_element_type=jnp.float32))
    EbT = jnp.exp2(lax.dot_general(ws, xb[:, _KGRP:2 * _KGRP], nt,
                                   preferred_element_type=jnp.float32))
    pieces = []
    for c in range(_HALF):
        pieces.append(lax.dot_general(cvs, EaT[c * _K:(c + 1) * _K, :], tn,
                                      preferred_element_type=jnp.float32))
    for c in range(_HALF):
        pieces.append(lax.dot_general(cvs, EbT[c * _K:(c + 1) * _K, :], tn,
                                      preferred_element_type=jnp.float32))
    yt = jnp.concatenate(pieces, axis=0)
    nd = lax.dot_general(yt, shuf_ref[...], tn,
                         preferred_element_type=jnp.float32)
    out_ref[...] = nd[:, 0:128] / nd[:, 128:256]


def _tc_main(x2, cktile, bd, wsraw, cva, shuf, gamma, beta):
    N20 = x2.shape[0]
    N = N20 * _D
    MS = 2048
    NBS = N20 // MS
    M20 = 1280
    NB = N20 // M20
    stats = functools.partial(_stats_body, inv_n=float(1.0 / N), nb=NBS)
    ws, cvs = pl.pallas_call(
        stats,
        grid=(NBS,),
        in_specs=[
            pl.BlockSpec((MS, _K), lambda j: (j, 0)),
            pl.BlockSpec((_K, _K), lambda j: (0, 0)),
            pl.BlockSpec((_K, _K), lambda j: (0, 0)),
            pl.BlockSpec((_WIDE, _KGRP), lambda j: (0, 0)),
            pl.BlockSpec((_K, _D), lambda j: (0, 0)),
            pl.BlockSpec((_K, 1), lambda j: (0, 0)),
            pl.BlockSpec((_K, 1), lambda j: (0, 0)),
        ],
        out_specs=[
            pl.BlockSpec((_WIDE, _KGRP), lambda j: (0, 0)),
            pl.BlockSpec((_K, _D), lambda j: (0, 0)),
        ],
        out_shape=[
            jax.ShapeDtypeStruct((_WIDE, _KGRP), jnp.bfloat16),
            jax.ShapeDtypeStruct((_K, _D), jnp.float32),
        ],
        scratch_shapes=[
            pltpu.VMEM((1, _K), jnp.float32),
            pltpu.VMEM((_K, _K), jnp.float32),
        ],
        compiler_params=pltpu.CompilerParams(
            dimension_semantics=("arbitrary",),
        ),
    )(x2, cktile, bd, wsraw, cva, gamma, beta)
    return pl.pallas_call(
        _apply_body,
        grid=(NB,),
        in_specs=[
            pl.BlockSpec((M20, _K), lambda j: (j, 0)),
            pl.BlockSpec((_WIDE, _KGRP), lambda j: (0, 0)),
            pl.BlockSpec((_K, _D), lambda j: (0, 0)),
            pl.BlockSpec((2 * _D * _D_OUT, 2 * _D * _D_OUT), lambda j: (0, 0)),
        ],
        out_specs=pl.BlockSpec((M20, _D * _D_OUT), lambda j: (j, 0)),
        out_shape=jax.ShapeDtypeStruct((N20, _D * _D_OUT), jnp.float32),
        compiler_params=pltpu.CompilerParams(
            dimension_semantics=("arbitrary",),
        ),
    )(x2, ws, cvs, shuf)


def _build_constants(centroids_k, centroids_v):
    ckT = centroids_k.T
    cktile = jnp.tile(ckT, (_D, 1))
    eye16 = jnp.eye(_D, dtype=jnp.float32)
    bd = jnp.kron(eye16, jnp.ones((_D_IN, _D_IN), jnp.float32))
    eye8 = jnp.eye(_HALF, dtype=jnp.float32)
    wsraw = jnp.reshape(
        jnp.einsum('ce,ki->ckei', eye8, centroids_k), (_WIDE, _KGRP))
    cva = jnp.concatenate(
        [centroids_v, jnp.ones((_K, 1), jnp.float32),
         jnp.zeros((_K, _D - _D_OUT - 1), jnp.float32)], axis=1)
    numpart = jnp.reshape(
        jnp.einsum('de,uv->duev', eye16, jnp.eye(_D, _D_OUT)),
        (_D * _D, _D * _D_OUT))
    denpart = jnp.reshape(
        jnp.einsum('de,u,v->duev', eye16,
                   (jnp.arange(_D) == _D_OUT).astype(jnp.float32),
                   jnp.ones((_D_OUT,), jnp.float32)),
        (_D * _D, _D * _D_OUT))
    shuf = jnp.concatenate([numpart, denpart], axis=1)
    return cktile, bd, wsraw, cva, shuf


def kernel(input, query_wemb, centroids_k, centroids_v, bn_gamma, bn_beta):
    idxs = jnp.reshape(input, (-1,))
    x2 = _sc_gather(query_wemb, idxs)
    cktile, bd, wsraw, cva, shuf = _build_constants(centroids_k, centroids_v)
    out128 = _tc_main(x2, cktile, bd, wsraw, cva, shuf,
                      jnp.reshape(bn_gamma, (_K, 1)),
                      jnp.reshape(bn_beta, (_K, 1)))
    out = jnp.reshape(out128, tuple(input.shape) + (_D * _D_OUT,))
    losses = jnp.zeros((), dtype=jnp.float32)
    return (out, losses)


# speedup vs baseline: 1.8414x; 1.0157x over previous
import functools

import jax
import jax.numpy as jnp
from jax import lax
from jax.experimental import pallas as pl
from jax.experimental.pallas import tpu as pltpu
from jax.experimental.pallas import tpu_sc as plsc

_D = 16
_D_IN = 32
_K = 512
_D_OUT = 8
_BN_EPS = 1e-3
_LOG2E = 1.4426950408889634
_HALF = _D // 2
_WIDE = _HALF * _K
_KGRP = _HALF * _D_IN


def _sc_gather(table, idx):
    B = idx.shape[0]
    Dw = table.shape[1]
    NW = 32
    b_per_w = B // NW
    C = 80
    n_chunks = b_per_w // C
    mesh = plsc.VectorSubcoreMesh(core_axis_name="c", subcore_axis_name="s")

    @functools.partial(
        pl.kernel,
        mesh=mesh,
        out_type=jax.ShapeDtypeStruct((B, Dw), jnp.float32),
        scratch_types=[
            pltpu.VMEM((C,), jnp.int32),
            pltpu.VMEM((C,), jnp.int32),
            pltpu.VMEM((C, Dw), jnp.float32),
            pltpu.VMEM((C, Dw), jnp.float32),
            pltpu.SemaphoreType.DMA,
            pltpu.SemaphoreType.DMA,
        ],
    )
    def k(table_hbm, idx_hbm, out_hbm, idx0, idx1, rows0, rows1, sem0, sem1):
        wid = lax.axis_index("s") * 2 + lax.axis_index("c")
        base = wid * b_per_w
        idxs, rows, sems = [idx0, idx1], [rows0, rows1], [sem0, sem1]
        handles = [None, None]
        pltpu.sync_copy(idx_hbm.at[pl.ds(base, C)], idxs[0])
        handles[0] = pltpu.async_copy(table_hbm.at[idxs[0]], rows[0], sems[0])
        for c in range(1, n_chunks + 1):
            b = c % 2
            if c < n_chunks:
                pltpu.sync_copy(idx_hbm.at[pl.ds(base + c * C, C)], idxs[b])
                handles[b] = pltpu.async_copy(
                    table_hbm.at[idxs[b]], rows[b], sems[b])
            pb = (c - 1) % 2
            handles[pb].wait()
            pltpu.sync_copy(rows[pb], out_hbm.at[pl.ds(base + (c - 1) * C, C)])

    return k(table, idx)


def _stats_body(x_ref, cktile_ref, bd_ref, wsraw_ref, cva_ref, g_ref, bt_ref,
                ws_ref, cvs_ref, cs_ref, p_ref, *, inv_n, nb):
    j = pl.program_id(0)
    xb = x_ref[...]
    cs = jnp.sum(xb, axis=0, keepdims=True)
    P = lax.dot_general(xb, xb, (((0,), (0,)), ((), ())),
                        preferred_element_type=jnp.float32)

    @pl.when(j == 0)
    def _():
        cs_ref[...] = cs
        p_ref[...] = P

    @pl.when(j > 0)
    def _():
        cs_ref[...] = cs_ref[...] + cs
        p_ref[...] = p_ref[...] + P

    @pl.when(j == nb - 1)
    def _():
        cktile = cktile_ref[...]
        mean = lax.dot_general(cktile, cs_ref[...], (((0,), (1,)), ((), ())),
                               preferred_element_type=jnp.float32) * inv_n
        pd = p_ref[...] * bd_ref[...]
        H = lax.dot_general(pd, cktile, (((1,), (0,)), ((), ())),
                            preferred_element_type=jnp.float32)
        ones_row = jnp.ones((1, _K), jnp.float32)
        ex2 = lax.dot_general(cktile * H, ones_row, (((0,), (1,)), ((), ())),
                              preferred_element_type=jnp.float32) * inv_n
        var = ex2 - mean * mean
        a_col = g_ref[...] * lax.rsqrt(var + _BN_EPS)
        b2_col = (bt_ref[...] - mean * a_col) * _LOG2E
        a2_col = a_col * _LOG2E
        cvs_ref[...] = cva_ref[...] * jnp.exp2(b2_col)
        a_t = jnp.concatenate([a2_col] * _HALF, axis=0)
        ws_ref[...] = (wsraw_ref[...] * a_t).astype(jnp.bfloat16)


def _apply_body(x_ref, ws_ref, cvs_ref, shuf_ref, out_ref):
    xb = x_ref[...].astype(jnp.bfloat16)
    ws = ws_ref[...]
    cvs = cvs_ref[...]
    nt = (((1,), (1,)), ((), ()))
    tn = (((0,), (0,)), ((), ()))
    EaT = jnp.exp2(lax.dot_general(ws, xb[:, 0:_KGRP], nt,
                                   preferred_element_type=jnp.float32))
    EbT = jnp.exp2(lax.dot_general(ws, xb[:, _KGRP:2 * _KGRP], nt,
                                   preferred_element_type=jnp.float32))
    pieces = []
    for c in range(_HALF):
        pieces.append(lax.dot_general(cvs, EaT[c * _K:(c + 1) * _K, :], tn,
                                      preferred_element_type=jnp.float32))
    for c in range(_HALF):
        pieces.append(lax.dot_general(cvs, EbT[c * _K:(c + 1) * _K, :], tn,
                                      preferred_element_type=jnp.float32))
    yt = jnp.concatenate(pieces, axis=0)
    nd = lax.dot_general(yt, shuf_ref[...], tn,
                         preferred_element_type=jnp.float32)
    out_ref[...] = nd[:, 0:128] / nd[:, 128:256]


def _tc_main(x2, cktile, bd, wsraw, cva, shuf, gamma, beta):
    N20 = x2.shape[0]
    N = N20 * _D
    MS = 2048
    NBS = N20 // MS
    M20 = 1280
    NB = N20 // M20
    stats = functools.partial(_stats_body, inv_n=float(1.0 / N), nb=NBS)
    ws, cvs = pl.pallas_call(
        stats,
        grid=(NBS,),
        in_specs=[
            pl.BlockSpec((MS, _K), lambda j: (j, 0)),
            pl.BlockSpec((_K, _K), lambda j: (0, 0)),
            pl.BlockSpec((_K, _K), lambda j: (0, 0)),
            pl.BlockSpec((_WIDE, _KGRP), lambda j: (0, 0)),
            pl.BlockSpec((_K, _D), lambda j: (0, 0)),
            pl.BlockSpec((_K, 1), lambda j: (0, 0)),
            pl.BlockSpec((_K, 1), lambda j: (0, 0)),
        ],
        out_specs=[
            pl.BlockSpec((_WIDE, _KGRP), lambda j: (0, 0)),
            pl.BlockSpec((_K, _D), lambda j: (0, 0)),
        ],
        out_shape=[
            jax.ShapeDtypeStruct((_WIDE, _KGRP), jnp.bfloat16),
            jax.ShapeDtypeStruct((_K, _D), jnp.float32),
        ],
        scratch_shapes=[
            pltpu.VMEM((1, _K), jnp.float32),
            pltpu.VMEM((_K, _K), jnp.float32),
        ],
        compiler_params=pltpu.CompilerParams(
            dimension_semantics=("arbitrary",),
        ),
    )(x2, cktile, bd, wsraw, cva, gamma, beta)
    return pl.pallas_call(
        _apply_body,
        grid=(NB,),
        in_specs=[
            pl.BlockSpec((M20, _K), lambda j: (j, 0)),
            pl.BlockSpec((_WIDE, _KGRP), lambda j: (0, 0)),
            pl.BlockSpec((_K, _D), lambda j: (0, 0)),
            pl.BlockSpec((2 * _D * _D_OUT, 2 * _D * _D_OUT), lambda j: (0, 0)),
        ],
        out_specs=pl.BlockSpec((M20, _D * _D_OUT), lambda j: (j, 0)),
        out_shape=jax.ShapeDtypeStruct((N20, _D * _D_OUT), jnp.float32),
        compiler_params=pltpu.CompilerParams(
            dimension_semantics=("arbitrary",),
        ),
    )(x2, ws, cvs, shuf)


def _build_constants(centroids_k, centroids_v):
    ckT = centroids_k.T
    cktile = jnp.tile(ckT, (_D, 1))
    eye16 = jnp.eye(_D, dtype=jnp.float32)
    bd = jnp.kron(eye16, jnp.ones((_D_IN, _D_IN), jnp.float32))
    eye8 = jnp.eye(_HALF, dtype=jnp.float32)
    wsraw = jnp.reshape(
        jnp.einsum('ce,ki->ckei', eye8, centroids_k), (_WIDE, _KGRP))
    cva = jnp.concatenate(
        [centroids_v, jnp.ones((_K, 1), jnp.float32),
         jnp.zeros((_K, _D - _D_OUT - 1), jnp.float32)], axis=1)
    numpart = jnp.reshape(
        jnp.einsum('de,uv->duev', eye16, jnp.eye(_D, _D_OUT)),
        (_D * _D, _D * _D_OUT))
    denpart = jnp.reshape(
        jnp.einsum('de,u,v->duev', eye16,
                   (jnp.arange(_D) == _D_OUT).astype(jnp.float32),
                   jnp.ones((_D_OUT,), jnp.float32)),
        (_D * _D, _D * _D_OUT))
    shuf = jnp.concatenate([numpart, denpart], axis=1)
    return cktile, bd, wsraw, cva, shuf


def kernel(input, query_wemb, centroids_k, centroids_v, bn_gamma, bn_beta):
    idxs = jnp.reshape(input, (-1,))
    x2 = _sc_gather(query_wemb, idxs)
    cktile, bd, wsraw, cva, shuf = _build_constants(centroids_k, centroids_v)
    out128 = _tc_main(x2, cktile, bd, wsraw, cva, shuf,
                      jnp.reshape(bn_gamma, (_K, 1)),
                      jnp.reshape(bn_beta, (_K, 1)))
    out = jnp.reshape(out128, tuple(input.shape) + (_D * _D_OUT,))
    losses = jnp.zeros((), dtype=jnp.float32)
    return (out, losses)
